# baseline (speedup 1.0000x reference)
import functools

import jax
import jax.numpy as jnp
from jax import lax
from jax.experimental import pallas as pl
from jax.experimental.pallas import tpu as pltpu
from jax.experimental.pallas import tpu_sc as plsc

N = 10000
E = 320000
D = 128

NC = 2
NS = 16
NW = NC * NS
EPW = E // NW
CH = 80
CPB = 8
NBLK = 16
EPWP = NBLK * CPB * CH
NP = N + 8
WSTRIDE = 624
WROWS = 640

_SC_MESH = plsc.VectorSubcoreMesh(core_axis_name="c", subcore_axis_name="s")


@functools.partial(
    pl.kernel,
    out_type=jax.ShapeDtypeStruct((NC, N, D), jnp.float32),
    mesh=_SC_MESH,
    scratch_types=[
        pltpu.VMEM((CPB, CH), jnp.int32),
        pltpu.VMEM((CPB, CH), jnp.int32),
        pltpu.VMEM((CPB, CH), jnp.int32),
        pltpu.VMEM((CPB, CH), jnp.int32),
        pltpu.VMEM((CH, D), jnp.float32),
        pltpu.VMEM((CH, D), jnp.float32),
        pltpu.VMEM_SHARED((N, D), jnp.float32),
        pltpu.SemaphoreType.DMA,
        pltpu.SemaphoreType.DMA,
        pltpu.SemaphoreType.DMA,
    ],
)
def _segsum(h_hbm, src_hbm, dst_hbm, out_hbm, sA, dA, sB, dB, rows0, rows1,
            acc_sh, semI, sem0, sem1):
    c = lax.axis_index("c")
    s = lax.axis_index("s")
    wid = s * NC + c
    row0 = jnp.minimum(s * WSTRIDE, N - WROWS)

    zero16 = jnp.zeros((16,), jnp.float32)

    def zrow(i, carry):
        for j in range(D // 16):
            rows0[i, pl.ds(j * 16, 16)] = zero16
        return carry

    lax.fori_loop(0, CH, zrow, 0)
    for k in range(WROWS // CH):
        pltpu.sync_copy(rows0, acc_sh.at[pl.ds(row0 + k * CH, CH)])
    plsc.subcore_barrier()

    def pipeline8(sbuf, dbuf):
        for k in range(CPB):
            pltpu.async_copy(h_hbm.at[sbuf.at[k]], rows0, sem0).wait()
            pltpu.sync_copy(rows0, acc_sh.at[dbuf.at[k]], add=True)

    pltpu.sync_copy(src_hbm.at[wid, 0], sA)
    pltpu.sync_copy(dst_hbm.at[wid, 0], dA)

    def block_pair(bb, carry):
        iB0 = pltpu.async_copy(src_hbm.at[wid, 2 * bb + 1], sB, semI)
        iB1 = pltpu.async_copy(dst_hbm.at[wid, 2 * bb + 1], dB, semI)
        pipeline8(sA, dA)
        iB0.wait()
        iB1.wait()
        nxt = jnp.minimum(2 * bb + 2, NBLK - 1)
        iA0 = pltpu.async_copy(src_hbm.at[wid, nxt], sA, semI)
        iA1 = pltpu.async_copy(dst_hbm.at[wid, nxt], dA, semI)
        pipeline8(sB, dB)
        iA0.wait()
        iA1.wait()
        return carry

    lax.fori_loop(0, NBLK // 2, block_pair, 0)
    plsc.subcore_barrier()

    for k in range(WROWS // CH):
        pltpu.sync_copy(acc_sh.at[pl.ds(row0 + k * CH, CH)], rows0)
        pltpu.sync_copy(rows0, out_hbm.at[c, pl.ds(row0 + k * CH, CH)])


def _bn_relu_combine(p_ref, h_ref, W_ref, b_ref, Wr_ref, br_ref, g_ref, be_ref):
    agg = p_ref[0] + p_ref[1]
    out = jnp.maximum(
        jnp.dot(agg, W_ref[...], preferred_element_type=jnp.float32)
        + b_ref[...], 0.0)
    res = jnp.maximum(
        jnp.dot(h_ref[...], Wr_ref[...], preferred_element_type=jnp.float32)
        + br_ref[...], 0.0)
    out = out + res
    mu = jnp.mean(out, axis=0, keepdims=True)
    var = jnp.mean((out - mu) ** 2, axis=0, keepdims=True)
    return g_ref[...] * (out - mu) * lax.rsqrt(var + 1e-5) + be_ref[...]


def _dense1_body(p_ref, h_ref, W_ref, b_ref, Wr_ref, br_ref, g_ref, be_ref,
                 o_ref):
    o_ref[...] = _bn_relu_combine(p_ref, h_ref, W_ref, b_ref, Wr_ref, br_ref,
                                  g_ref, be_ref)


def _dense2_body(p_ref, h_ref, W_ref, b_ref, Wr_ref, br_ref, g_ref, be_ref,
                 watt_ref, batt_ref, o_ref):
    h2 = _bn_relu_combine(p_ref, h_ref, W_ref, b_ref, Wr_ref, br_ref, g_ref,
                          be_ref)
    logit = jnp.sum(h2 * watt_ref[...], axis=1, keepdims=True) + batt_ref[...]
    wgt = 1.0 / (1.0 + jnp.exp(-logit))
    hsum = jnp.sum(wgt * h2, axis=0, keepdims=True)
    hmax = jnp.max(h2, axis=0, keepdims=True)
    o_ref[...] = jnp.concatenate([hsum, hmax], axis=1)


_dense1 = pl.pallas_call(
    _dense1_body,
    out_shape=jax.ShapeDtypeStruct((N, D), jnp.float32),
)

_dense2 = pl.pallas_call(
    _dense2_body,
    out_shape=jax.ShapeDtypeStruct((1, 2 * D), jnp.float32),
)


def kernel(x, edge_index, W1, b1, Wr1, br1, g1, be1, W2, b2, Wr2, br2, g2,
           be2, w_att, b_att):
    pad = ((0, 0), (0, EPWP - EPW))
    src = jnp.pad(edge_index[0].reshape(NW, EPW), pad,
                  constant_values=N).reshape(NW, NBLK, CPB, CH)
    dpad = (jnp.arange(NW * (EPWP - EPW), dtype=jnp.int32) % N).reshape(
        NW, EPWP - EPW)
    dst = jnp.concatenate([edge_index[1].reshape(NW, EPW), dpad],
                          axis=1).reshape(NW, NBLK, CPB, CH)
    rowpad = ((0, NP - N), (0, 0))
    row = lambda v: v.reshape(1, -1)
    P1 = _segsum(jnp.pad(x, rowpad), src, dst)
    h1 = _dense1(P1, x, W1, row(b1), Wr1, row(br1), row(g1), row(be1))
    P2 = _segsum(jnp.pad(h1, rowpad), src, dst)
    return _dense2(P2, h1, W2, row(b2), Wr2, row(br2), row(g2), row(be2),
                   row(w_att), b_att.reshape(1, 1))


# speedup vs baseline: 1.7374x; 1.7374x over previous
import functools

import jax
import jax.numpy as jnp
from jax import lax
from jax.experimental import pallas as pl
from jax.experimental.pallas import tpu as pltpu
from jax.experimental.pallas import tpu_sc as plsc

N = 10000
E = 320000
D = 128

NC = 2
NS = 16
NW = NC * NS
EPW = E // NW
CH = 64
NHALF = 79
NCHUNK = 2 * NHALF
EPWP = NCHUNK * CH
NP = N + 8
WSTRIDE = 624
WROWS = 640

_SC_MESH = plsc.VectorSubcoreMesh(core_axis_name="c", subcore_axis_name="s")


@functools.partial(
    pl.kernel,
    out_type=jax.ShapeDtypeStruct((NC, N, D), jnp.float32),
    mesh=_SC_MESH,
    scratch_types=[
        pltpu.VMEM((NHALF, CH), jnp.int32),
        pltpu.VMEM((NHALF, CH), jnp.int32),
        pltpu.VMEM((CH, D), jnp.float32),
        pltpu.VMEM((CH, D), jnp.float32),
        pltpu.VMEM_SHARED((N, D), jnp.float32),
        pltpu.SemaphoreType.DMA,
        pltpu.SemaphoreType.DMA,
    ],
)
def _segsum(h_hbm, src_hbm, dst_hbm, out_hbm, src_all, dst_all, rows0, rows1,
            acc_sh, sem0, sem1):
    c = lax.axis_index("c")
    s = lax.axis_index("s")
    wid = s * NC + c
    row0 = jnp.minimum(s * WSTRIDE, N - WROWS)

    zero16 = jnp.zeros((16,), jnp.float32)

    def zrow(i, carry):
        for j in range(D // 16):
            rows0[i, pl.ds(j * 16, 16)] = zero16
        return carry

    lax.fori_loop(0, CH, zrow, 0)
    for k in range(WROWS // CH):
        pltpu.sync_copy(rows0, acc_sh.at[pl.ds(row0 + k * CH, CH)])
    plsc.subcore_barrier()

    def chunk_pair(i, carry):
        j0 = 2 * i
        d1 = pltpu.async_copy(h_hbm.at[src_all.at[j0 + 1]], rows1, sem1)
        pltpu.sync_copy(rows0, acc_sh.at[dst_all.at[j0]], add=True)
        d0 = pltpu.async_copy(h_hbm.at[src_all.at[j0 + 2]], rows0, sem0)
        d1.wait()
        pltpu.sync_copy(rows1, acc_sh.at[dst_all.at[j0 + 1]], add=True)
        d0.wait()
        return carry

    for half in range(2):
        pltpu.sync_copy(src_hbm.at[wid, half], src_all)
        pltpu.sync_copy(dst_hbm.at[wid, half], dst_all)
        pltpu.async_copy(h_hbm.at[src_all.at[0]], rows0, sem0).wait()
        lax.fori_loop(0, (NHALF - 1) // 2, chunk_pair, 0)
        pltpu.sync_copy(rows0, acc_sh.at[dst_all.at[NHALF - 1]], add=True)
    plsc.subcore_barrier()

    for k in range(WROWS // CH):
        pltpu.sync_copy(acc_sh.at[pl.ds(row0 + k * CH, CH)], rows0)
        pltpu.sync_copy(rows0, out_hbm.at[c, pl.ds(row0 + k * CH, CH)])


def _bn_relu_combine(p_ref, h_ref, W_ref, b_ref, Wr_ref, br_ref, g_ref, be_ref):
    agg = p_ref[0] + p_ref[1]
    out = jnp.maximum(
        jnp.dot(agg, W_ref[...], preferred_element_type=jnp.float32)
        + b_ref[...], 0.0)
    res = jnp.maximum(
        jnp.dot(h_ref[...], Wr_ref[...], preferred_element_type=jnp.float32)
        + br_ref[...], 0.0)
    out = out + res
    mu = jnp.mean(out, axis=0, keepdims=True)
    var = jnp.mean((out - mu) ** 2, axis=0, keepdims=True)
    return g_ref[...] * (out - mu) * lax.rsqrt(var + 1e-5) + be_ref[...]


def _dense1_body(p_ref, h_ref, W_ref, b_ref, Wr_ref, br_ref, g_ref, be_ref,
                 o_ref):
    o_ref[...] = _bn_relu_combine(p_ref, h_ref, W_ref, b_ref, Wr_ref, br_ref,
                                  g_ref, be_ref)


def _dense2_body(p_ref, h_ref, W_ref, b_ref, Wr_ref, br_ref, g_ref, be_ref,
                 watt_ref, batt_ref, o_ref):
    h2 = _bn_relu_combine(p_ref, h_ref, W_ref, b_ref, Wr_ref, br_ref, g_ref,
                          be_ref)
    logit = jnp.sum(h2 * watt_ref[...], axis=1, keepdims=True) + batt_ref[...]
    wgt = 1.0 / (1.0 + jnp.exp(-logit))
    hsum = jnp.sum(wgt * h2, axis=0, keepdims=True)
    hmax = jnp.max(h2, axis=0, keepdims=True)
    o_ref[...] = jnp.concatenate([hsum, hmax], axis=1)


_dense1 = pl.pallas_call(
    _dense1_body,
    out_shape=jax.ShapeDtypeStruct((N, D), jnp.float32),
)

_dense2 = pl.pallas_call(
    _dense2_body,
    out_shape=jax.ShapeDtypeStruct((1, 2 * D), jnp.float32),
)


def kernel(x, edge_index, W1, b1, Wr1, br1, g1, be1, W2, b2, Wr2, br2, g2,
           be2, w_att, b_att):
    pad = ((0, 0), (0, EPWP - EPW))
    src = jnp.pad(edge_index[0].reshape(NW, EPW), pad,
                  constant_values=N).reshape(NW, 2, NHALF, CH)
    dpad = (jnp.arange(NW * (EPWP - EPW), dtype=jnp.int32) % N).reshape(
        NW, EPWP - EPW)
    dst = jnp.concatenate([edge_index[1].reshape(NW, EPW), dpad],
                          axis=1).reshape(NW, 2, NHALF, CH)
    rowpad = ((0, NP - N), (0, 0))
    row = lambda v: v.reshape(1, -1)
    P1 = _segsum(jnp.pad(x, rowpad), src, dst)
    h1 = _dense1(P1, x, W1, row(b1), Wr1, row(br1), row(g1), row(be1))
    P2 = _segsum(jnp.pad(h1, rowpad), src, dst)
    return _dense2(P2, h1, W2, row(b2), Wr2, row(br2), row(g2), row(be2),
                   row(w_att), b_att.reshape(1, 1))


# speedup vs baseline: 2.8655x; 1.6493x over previous
import functools

import jax
import jax.numpy as jnp
from jax import lax
from jax.experimental import pallas as pl
from jax.experimental.pallas import tpu as pltpu
from jax.experimental.pallas import tpu_sc as plsc

N = 10000
E = 320000
D = 128

NC = 2
NS = 16
NW = NC * NS
EPW = E // NW
CH = 64
NHALF = 79
NCHUNK = 2 * NHALF
EPWP = NCHUNK * CH
NP = N + 8
WSTRIDE = 624
WROWS = 640

_SC_MESH = plsc.VectorSubcoreMesh(core_axis_name="c", subcore_axis_name="s")


@functools.partial(
    pl.kernel,
    out_type=jax.ShapeDtypeStruct((NC, N, D), jnp.float32),
    mesh=_SC_MESH,
    scratch_types=[
        pltpu.VMEM((NHALF, CH), jnp.int32),
        pltpu.VMEM((NHALF, CH), jnp.int32),
        pltpu.VMEM((CH, D), jnp.float32),
        pltpu.VMEM((CH, D), jnp.float32),
        pltpu.VMEM_SHARED((N, D), jnp.float32),
        pltpu.SemaphoreType.DMA,
        pltpu.SemaphoreType.DMA,
    ],
)
def _segsum(h_hbm, src_hbm, dst_hbm, out_hbm, src_all, dst_all, rows0, rows1,
            acc_sh, sem0, sem1):
    c = lax.axis_index("c")
    s = lax.axis_index("s")
    wid = s * NC + c
    row0 = jnp.minimum(s * WSTRIDE, N - WROWS)

    zero16 = jnp.zeros((16,), jnp.float32)

    def zrow(i, carry):
        for j in range(D // 16):
            rows0[i, pl.ds(j * 16, 16)] = zero16
        return carry

    lax.fori_loop(0, CH, zrow, 0)
    for k in range(WROWS // CH):
        pltpu.sync_copy(rows0, acc_sh.at[pl.ds(row0 + k * CH, CH)])
    plsc.subcore_barrier()

    def chunk_pair(i, carry):
        j0 = 2 * i
        d1 = pltpu.async_copy(h_hbm.at[src_all.at[j0 + 1]], rows1, sem1)
        pltpu.sync_copy(rows0, acc_sh.at[dst_all.at[j0]], add=True)
        d0 = pltpu.async_copy(h_hbm.at[src_all.at[j0 + 2]], rows0, sem0)
        d1.wait()
        pltpu.sync_copy(rows1, acc_sh.at[dst_all.at[j0 + 1]], add=True)
        d0.wait()
        return carry

    for half in range(2):
        pltpu.sync_copy(src_hbm.at[wid, half], src_all)
        pltpu.sync_copy(dst_hbm.at[wid, half], dst_all)
        pltpu.async_copy(h_hbm.at[src_all.at[0]], rows0, sem0).wait()
        lax.fori_loop(0, (NHALF - 1) // 2, chunk_pair, 0)
        pltpu.sync_copy(rows0, acc_sh.at[dst_all.at[NHALF - 1]], add=True)
    plsc.subcore_barrier()

    for k in range(WROWS // CH):
        pltpu.sync_copy(acc_sh.at[pl.ds(row0 + k * CH, CH)], rows0)
        pltpu.sync_copy(rows0, out_hbm.at[c, pl.ds(row0 + k * CH, CH)])


def _bn_relu_combine(p_ref, h_ref, W_ref, b_ref, Wr_ref, br_ref, g_ref, be_ref):
    agg = p_ref[0] + p_ref[1]
    out = jnp.maximum(
        jnp.dot(agg, W_ref[...], preferred_element_type=jnp.float32)
        + b_ref[...], 0.0)
    res = jnp.maximum(
        jnp.dot(h_ref[...], Wr_ref[...], preferred_element_type=jnp.float32)
        + br_ref[...], 0.0)
    out = out + res
    mu = jnp.mean(out, axis=0, keepdims=True)
    var = jnp.mean((out - mu) ** 2, axis=0, keepdims=True)
    return g_ref[...] * (out - mu) * lax.rsqrt(var + 1e-5) + be_ref[...]


def _dense1_body(p_ref, h_ref, W_ref, b_ref, Wr_ref, br_ref, g_ref, be_ref,
                 o_ref):
    o_ref[...] = _bn_relu_combine(p_ref, h_ref, W_ref, b_ref, Wr_ref, br_ref,
                                  g_ref, be_ref)


def _dense2_body(p_ref, h_ref, W_ref, b_ref, Wr_ref, br_ref, g_ref, be_ref,
                 watt_ref, batt_ref, o_ref):
    h2 = _bn_relu_combine(p_ref, h_ref, W_ref, b_ref, Wr_ref, br_ref, g_ref,
                          be_ref)
    logit = jnp.sum(h2 * watt_ref[...], axis=1, keepdims=True) + batt_ref[...]
    wgt = 1.0 / (1.0 + jnp.exp(-logit))
    hsum = jnp.sum(wgt * h2, axis=0, keepdims=True)
    hmax = jnp.max(h2, axis=0, keepdims=True)
    o_ref[...] = jnp.concatenate([hsum, hmax], axis=1)


_dense1 = pl.pallas_call(
    _dense1_body,
    out_shape=jax.ShapeDtypeStruct((N, D), jnp.float32),
)

_dense2 = pl.pallas_call(
    _dense2_body,
    out_shape=jax.ShapeDtypeStruct((1, 2 * D), jnp.float32),
)


def kernel(x, edge_index, W1, b1, Wr1, br1, g1, be1, W2, b2, Wr2, br2, g2,
           be2, w_att, b_att):
    spad = (N + (jnp.arange(NW * (EPWP - EPW), dtype=jnp.int32)
                 % (NP - N))).reshape(NW, EPWP - EPW)
    src = jnp.concatenate([edge_index[0].reshape(NW, EPW), spad],
                          axis=1).reshape(NW, 2, NHALF, CH)
    dpad = (jnp.arange(NW * (EPWP - EPW), dtype=jnp.int32) % N).reshape(
        NW, EPWP - EPW)
    dst = jnp.concatenate([edge_index[1].reshape(NW, EPW), dpad],
                          axis=1).reshape(NW, 2, NHALF, CH)
    rowpad = ((0, NP - N), (0, 0))
    row = lambda v: v.reshape(1, -1)
    P1 = _segsum(jnp.pad(x, rowpad), src, dst)
    h1 = _dense1(P1, x, W1, row(b1), Wr1, row(br1), row(g1), row(be1))
    P2 = _segsum(jnp.pad(h1, rowpad), src, dst)
    return _dense2(P2, h1, W2, row(b2), Wr2, row(br2), row(g2), row(be2),
                   row(w_att), b_att.reshape(1, 1))


# speedup vs baseline: 3.0485x; 1.0639x over previous
import functools

import jax
import jax.numpy as jnp
from jax import lax
from jax.experimental import pallas as pl
from jax.experimental.pallas import tpu as pltpu
from jax.experimental.pallas import tpu_sc as plsc

N = 10000
E = 320000
D = 128

NC = 2
NS = 16
NW = NC * NS
EPW = E // NW
CH = 64
NHALF = 79
NCHUNK = 2 * NHALF
EPWP = NCHUNK * CH
NP = N + 128
WSTRIDE = 624
WROWS = 640

_SC_MESH = plsc.VectorSubcoreMesh(core_axis_name="c", subcore_axis_name="s")


@functools.partial(
    pl.kernel,
    out_type=jax.ShapeDtypeStruct((NC, N, D), jnp.float32),
    mesh=_SC_MESH,
    scratch_types=[
        pltpu.VMEM((NHALF, CH), jnp.int32),
        pltpu.VMEM((NHALF, CH), jnp.int32),
        pltpu.VMEM((CH, D), jnp.float32),
        pltpu.VMEM((CH, D), jnp.float32),
        pltpu.VMEM_SHARED((N, D), jnp.float32),
        pltpu.SemaphoreType.DMA,
        pltpu.SemaphoreType.DMA,
    ],
)
def _segsum(h_hbm, src_hbm, dst_hbm, out_hbm, src_all, dst_all, rows0, rows1,
            acc_sh, sem0, sem1):
    c = lax.axis_index("c")
    s = lax.axis_index("s")
    wid = s * NC + c
    row0 = jnp.minimum(s * WSTRIDE, N - WROWS)

    zero16 = jnp.zeros((16,), jnp.float32)

    def zrow(i, carry):
        for j in range(D // 16):
            rows0[i, pl.ds(j * 16, 16)] = zero16
        return carry

    lax.fori_loop(0, CH, zrow, 0)
    for k in range(WROWS // CH):
        pltpu.sync_copy(rows0, acc_sh.at[pl.ds(row0 + k * CH, CH)])
    plsc.subcore_barrier()

    def chunk_pair(i, carry):
        j0 = 2 * i
        d1 = pltpu.async_copy(h_hbm.at[src_all.at[j0 + 1]], rows1, sem1)
        pltpu.sync_copy(rows0, acc_sh.at[dst_all.at[j0]], add=True)
        d0 = pltpu.async_copy(h_hbm.at[src_all.at[j0 + 2]], rows0, sem0)
        d1.wait()
        pltpu.sync_copy(rows1, acc_sh.at[dst_all.at[j0 + 1]], add=True)
        d0.wait()
        return carry

    for half in range(2):
        pltpu.sync_copy(src_hbm.at[wid, half], src_all)
        pltpu.sync_copy(dst_hbm.at[wid, half], dst_all)
        pltpu.async_copy(h_hbm.at[src_all.at[0]], rows0, sem0).wait()
        lax.fori_loop(0, (NHALF - 1) // 2, chunk_pair, 0)
        pltpu.sync_copy(rows0, acc_sh.at[dst_all.at[NHALF - 1]], add=True)
    plsc.subcore_barrier()

    for k in range(WROWS // CH):
        pltpu.sync_copy(acc_sh.at[pl.ds(row0 + k * CH, CH)], rows0)
        pltpu.sync_copy(rows0, out_hbm.at[c, pl.ds(row0 + k * CH, CH)])


def _bn_relu_combine(p_ref, h_ref, W_ref, b_ref, Wr_ref, br_ref, g_ref, be_ref):
    agg = p_ref[0] + p_ref[1]
    out = jnp.maximum(
        jnp.dot(agg, W_ref[...], preferred_element_type=jnp.float32)
        + b_ref[...], 0.0)
    res = jnp.maximum(
        jnp.dot(h_ref[...], Wr_ref[...], preferred_element_type=jnp.float32)
        + br_ref[...], 0.0)
    out = out + res
    mu = jnp.mean(out, axis=0, keepdims=True)
    var = jnp.mean((out - mu) ** 2, axis=0, keepdims=True)
    return g_ref[...] * (out - mu) * lax.rsqrt(var + 1e-5) + be_ref[...]


def _dense1_body(p_ref, h_ref, W_ref, b_ref, Wr_ref, br_ref, g_ref, be_ref,
                 o_ref):
    o_ref[...] = _bn_relu_combine(p_ref, h_ref, W_ref, b_ref, Wr_ref, br_ref,
                                  g_ref, be_ref)


def _dense2_body(p_ref, h_ref, W_ref, b_ref, Wr_ref, br_ref, g_ref, be_ref,
                 watt_ref, batt_ref, o_ref):
    h2 = _bn_relu_combine(p_ref, h_ref, W_ref, b_ref, Wr_ref, br_ref, g_ref,
                          be_ref)
    logit = jnp.sum(h2 * watt_ref[...], axis=1, keepdims=True) + batt_ref[...]
    wgt = 1.0 / (1.0 + jnp.exp(-logit))
    hsum = jnp.sum(wgt * h2, axis=0, keepdims=True)
    hmax = jnp.max(h2, axis=0, keepdims=True)
    o_ref[...] = jnp.concatenate([hsum, hmax], axis=1)


_dense1 = pl.pallas_call(
    _dense1_body,
    out_shape=jax.ShapeDtypeStruct((N, D), jnp.float32),
)

_dense2 = pl.pallas_call(
    _dense2_body,
    out_shape=jax.ShapeDtypeStruct((1, 2 * D), jnp.float32),
)


def kernel(x, edge_index, W1, b1, Wr1, br1, g1, be1, W2, b2, Wr2, br2, g2,
           be2, w_att, b_att):
    spad = (N + (jnp.arange(NW * (EPWP - EPW), dtype=jnp.int32)
                 % (NP - N))).reshape(NW, EPWP - EPW)
    src = jnp.concatenate([edge_index[0].reshape(NW, EPW), spad],
                          axis=1).reshape(NW, 2, NHALF, CH)
    dpad = (jnp.arange(NW * (EPWP - EPW), dtype=jnp.int32) % N).reshape(
        NW, EPWP - EPW)
    dst = jnp.concatenate([edge_index[1].reshape(NW, EPW), dpad],
                          axis=1).reshape(NW, 2, NHALF, CH)
    rowpad = ((0, NP - N), (0, 0))
    row = lambda v: v.reshape(1, -1)
    P1 = _segsum(jnp.pad(x, rowpad), src, dst)
    h1 = _dense1(P1, x, W1, row(b1), Wr1, row(br1), row(g1), row(be1))
    P2 = _segsum(jnp.pad(h1, rowpad), src, dst)
    return _dense2(P2, h1, W2, row(b2), Wr2, row(br2), row(g2), row(be2),
                   row(w_att), b_att.reshape(1, 1))


# speedup vs baseline: 3.0797x; 1.0102x over previous
import functools

import jax
import jax.numpy as jnp
from jax import lax
from jax.experimental import pallas as pl
from jax.experimental.pallas import tpu as pltpu
from jax.experimental.pallas import tpu_sc as plsc

N = 10000
E = 320000
D = 128

NC = 2
NS = 16
NW = NC * NS
EPW = E // NW
CH = 64
NHALF = 79
NCHUNK = 2 * NHALF
EPWP = NCHUNK * CH
NP = N + 128
WSTRIDE = 624
WROWS = 640

_SC_MESH = plsc.VectorSubcoreMesh(core_axis_name="c", subcore_axis_name="s")


@functools.partial(
    pl.kernel,
    out_type=jax.ShapeDtypeStruct((NC, N, D), jnp.float32),
    mesh=_SC_MESH,
    scratch_types=[
        pltpu.VMEM((NHALF, CH), jnp.int32),
        pltpu.VMEM((NHALF, CH), jnp.int32),
        pltpu.VMEM((CH, D), jnp.float32),
        pltpu.VMEM((CH, D), jnp.float32),
        pltpu.VMEM_SHARED((N, D), jnp.float32),
        pltpu.SemaphoreType.DMA,
        pltpu.SemaphoreType.DMA,
    ],
)
def _segsum(h_hbm, src_hbm, dst_hbm, out_hbm, src_all, dst_all, rows0, rows1,
            acc_sh, sem0, sem1):
    c = lax.axis_index("c")
    s = lax.axis_index("s")
    wid = s * NC + c
    row0 = jnp.minimum(s * WSTRIDE, N - WROWS)

    zero16 = jnp.zeros((16,), jnp.float32)

    def zrow(i, carry):
        for j in range(D // 16):
            rows0[i, pl.ds(j * 16, 16)] = zero16
        return carry

    lax.fori_loop(0, CH, zrow, 0)
    for k in range(WROWS // CH):
        pltpu.sync_copy(rows0, acc_sh.at[pl.ds(row0 + k * CH, CH)])
    plsc.subcore_barrier()

    def chunk_pair(i, carry):
        j0 = 2 * i
        d1 = pltpu.async_copy(h_hbm.at[src_all.at[j0 + 1]], rows1, sem1)
        pltpu.sync_copy(rows0, acc_sh.at[dst_all.at[j0]], add=True)
        d0 = pltpu.async_copy(h_hbm.at[src_all.at[j0 + 2]], rows0, sem0)
        d1.wait()
        pltpu.sync_copy(rows1, acc_sh.at[dst_all.at[j0 + 1]], add=True)
        d0.wait()
        return carry

    for half in range(2):
        pltpu.sync_copy(src_hbm.at[wid, half], src_all)
        pltpu.sync_copy(dst_hbm.at[wid, half], dst_all)
        pltpu.async_copy(h_hbm.at[src_all.at[0]], rows0, sem0).wait()
        lax.fori_loop(0, (NHALF - 1) // 2, chunk_pair, 0)
        pltpu.sync_copy(rows0, acc_sh.at[dst_all.at[NHALF - 1]], add=True)
    plsc.subcore_barrier()

    for k in range(WROWS // CH):
        pltpu.sync_copy(acc_sh.at[pl.ds(row0 + k * CH, CH)], rows0)
        pltpu.sync_copy(rows0, out_hbm.at[c, pl.ds(row0 + k * CH, CH)])


def _bn_relu_combine(p_ref, h_ref, W_ref, b_ref, Wr_ref, br_ref, g_ref, be_ref):
    agg = p_ref[0] + p_ref[1]
    out = jnp.maximum(
        jnp.dot(agg, W_ref[...], preferred_element_type=jnp.float32)
        + b_ref[...], 0.0)
    res = jnp.maximum(
        jnp.dot(h_ref[...], Wr_ref[...], preferred_element_type=jnp.float32)
        + br_ref[...], 0.0)
    out = out + res
    mu = jnp.mean(out, axis=0, keepdims=True)
    var = jnp.mean((out - mu) ** 2, axis=0, keepdims=True)
    return g_ref[...] * (out - mu) * lax.rsqrt(var + 1e-5) + be_ref[...]


def _dense1_body(p_ref, h_ref, W_ref, b_ref, Wr_ref, br_ref, g_ref, be_ref,
                 o_ref):
    o_ref[pl.ds(0, N), :] = _bn_relu_combine(p_ref, h_ref, W_ref, b_ref,
                                             Wr_ref, br_ref, g_ref, be_ref)
    o_ref[pl.ds(N, NP - N), :] = jnp.zeros((NP - N, D), jnp.float32)


def _dense2_body(p_ref, h_ref, W_ref, b_ref, Wr_ref, br_ref, g_ref, be_ref,
                 watt_ref, batt_ref, o_ref):
    h2 = _bn_relu_combine(p_ref, h_ref.at[pl.ds(0, N), :], W_ref, b_ref,
                          Wr_ref, br_ref, g_ref, be_ref)
    logit = jnp.sum(h2 * watt_ref[...], axis=1, keepdims=True) + batt_ref[...]
    wgt = 1.0 / (1.0 + jnp.exp(-logit))
    hsum = jnp.sum(wgt * h2, axis=0, keepdims=True)
    hmax = jnp.max(h2, axis=0, keepdims=True)
    o_ref[...] = jnp.concatenate([hsum, hmax], axis=1)


_dense1 = pl.pallas_call(
    _dense1_body,
    out_shape=jax.ShapeDtypeStruct((NP, D), jnp.float32),
)

_dense2 = pl.pallas_call(
    _dense2_body,
    out_shape=jax.ShapeDtypeStruct((1, 2 * D), jnp.float32),
)


def kernel(x, edge_index, W1, b1, Wr1, br1, g1, be1, W2, b2, Wr2, br2, g2,
           be2, w_att, b_att):
    spad = (N + (jnp.arange(NW * (EPWP - EPW), dtype=jnp.int32)
                 % (NP - N))).reshape(NW, EPWP - EPW)
    src = jnp.concatenate([edge_index[0].reshape(NW, EPW), spad],
                          axis=1).reshape(NW, 2, NHALF, CH)
    dpad = (jnp.arange(NW * (EPWP - EPW), dtype=jnp.int32) % N).reshape(
        NW, EPWP - EPW)
    dst = jnp.concatenate([edge_index[1].reshape(NW, EPW), dpad],
                          axis=1).reshape(NW, 2, NHALF, CH)
    rowpad = ((0, NP - N), (0, 0))
    row = lambda v: v.reshape(1, -1)
    P1 = _segsum(jnp.pad(x, rowpad), src, dst)
    h1 = _dense1(P1, x, W1, row(b1), Wr1, row(br1), row(g1), row(be1))
    P2 = _segsum(h1, src, dst)
    return _dense2(P2, h1, W2, row(b2), Wr2, row(br2), row(g2), row(be2),
                   row(w_att), b_att.reshape(1, 1))


# speedup vs baseline: 3.1565x; 1.0250x over previous
import functools

import jax
import jax.numpy as jnp
from jax import lax
from jax.experimental import pallas as pl
from jax.experimental.pallas import tpu as pltpu
from jax.experimental.pallas import tpu_sc as plsc

N = 10000
E = 320000
D = 128

NC = 2
NS = 16
NW = NC * NS
EPW = E // NW
CH = 64
NHALF = 79
NCHUNK = 2 * NHALF
EPWP = NCHUNK * CH
NP = N + 128
WSTRIDE = 624
WROWS = 640

_SC_MESH = plsc.VectorSubcoreMesh(core_axis_name="c", subcore_axis_name="s")


@functools.partial(
    pl.kernel,
    out_type=jax.ShapeDtypeStruct((NC, N, D), jnp.float32),
    mesh=_SC_MESH,
    scratch_types=[
        pltpu.VMEM((NHALF, CH), jnp.int32),
        pltpu.VMEM((NHALF, CH), jnp.int32),
        pltpu.VMEM((CH, D), jnp.float32),
        pltpu.VMEM((CH, D), jnp.float32),
        pltpu.VMEM_SHARED((N, D), jnp.float32),
        pltpu.SemaphoreType.DMA,
        pltpu.SemaphoreType.DMA,
    ],
)
def _segsum(h_hbm, src_hbm, dst_hbm, out_hbm, src_all, dst_all, rows0, rows1,
            acc_sh, sem0, sem1):
    c = lax.axis_index("c")
    s = lax.axis_index("s")
    wid = s * NC + c
    row0 = jnp.minimum(s * WSTRIDE, N - WROWS)

    zero16 = jnp.zeros((16,), jnp.float32)

    def zrow(i, carry):
        for j in range(D // 16):
            rows0[i, pl.ds(j * 16, 16)] = zero16
        return carry

    lax.fori_loop(0, CH, zrow, 0)
    for k in range(WROWS // CH):
        pltpu.sync_copy(rows0, acc_sh.at[pl.ds(row0 + k * CH, CH)])
    plsc.subcore_barrier()

    def chunk_pair(i, carry):
        j0 = 2 * i
        d1 = pltpu.async_copy(h_hbm.at[src_all.at[j0 + 1]], rows1, sem1)
        pltpu.sync_copy(rows0, acc_sh.at[dst_all.at[j0]], add=True)
        d0 = pltpu.async_copy(h_hbm.at[src_all.at[j0 + 2]], rows0, sem0)
        d1.wait()
        pltpu.sync_copy(rows1, acc_sh.at[dst_all.at[j0 + 1]], add=True)
        d0.wait()
        return carry

    for half in range(2):
        pltpu.sync_copy(src_hbm.at[wid, half], src_all)
        pltpu.sync_copy(dst_hbm.at[wid, half], dst_all)
        pltpu.async_copy(h_hbm.at[src_all.at[0]], rows0, sem0).wait()
        lax.fori_loop(0, (NHALF - 1) // 2, chunk_pair, 0)
        pltpu.sync_copy(rows0, acc_sh.at[dst_all.at[NHALF - 1]], add=True)
    plsc.subcore_barrier()

    for k in range(WROWS // CH):
        pltpu.sync_copy(acc_sh.at[pl.ds(row0 + k * CH, CH)], rows0)
        pltpu.sync_copy(rows0, out_hbm.at[c, pl.ds(row0 + k * CH, CH)])


def _res_body(h_ref, Wr_ref, br_ref, o_ref):
    o_ref[...] = jnp.maximum(
        jnp.dot(h_ref[pl.ds(0, N), :], Wr_ref[...],
                preferred_element_type=jnp.float32) + br_ref[...], 0.0)


def _bn_combine(p_ref, res_ref, W_ref, b_ref, g_ref, be_ref):
    agg = p_ref[0] + p_ref[1]
    out = jnp.maximum(
        jnp.dot(agg, W_ref[...], preferred_element_type=jnp.float32)
        + b_ref[...], 0.0)
    out = out + res_ref[...]
    mu = jnp.mean(out, axis=0, keepdims=True)
    var = jnp.mean((out - mu) ** 2, axis=0, keepdims=True)
    return g_ref[...] * (out - mu) * lax.rsqrt(var + 1e-5) + be_ref[...]


def _comb1_body(p_ref, res_ref, W_ref, b_ref, g_ref, be_ref, o_ref):
    o_ref[pl.ds(0, N), :] = _bn_combine(p_ref, res_ref, W_ref, b_ref, g_ref,
                                        be_ref)
    o_ref[pl.ds(N, NP - N), :] = jnp.zeros((NP - N, D), jnp.float32)


def _comb2_body(p_ref, res_ref, W_ref, b_ref, g_ref, be_ref, watt_ref,
                batt_ref, o_ref):
    h2 = _bn_combine(p_ref, res_ref, W_ref, b_ref, g_ref, be_ref)
    logit = jnp.sum(h2 * watt_ref[...], axis=1, keepdims=True) + batt_ref[...]
    wgt = 1.0 / (1.0 + jnp.exp(-logit))
    hsum = jnp.sum(wgt * h2, axis=0, keepdims=True)
    hmax = jnp.max(h2, axis=0, keepdims=True)
    o_ref[...] = jnp.concatenate([hsum, hmax], axis=1)


_resk = pl.pallas_call(
    _res_body,
    out_shape=jax.ShapeDtypeStruct((N, D), jnp.float32),
)

_comb1 = pl.pallas_call(
    _comb1_body,
    out_shape=jax.ShapeDtypeStruct((NP, D), jnp.float32),
)

_comb2 = pl.pallas_call(
    _comb2_body,
    out_shape=jax.ShapeDtypeStruct((1, 2 * D), jnp.float32),
)


def kernel(x, edge_index, W1, b1, Wr1, br1, g1, be1, W2, b2, Wr2, br2, g2,
           be2, w_att, b_att):
    spad = (N + (jnp.arange(NW * (EPWP - EPW), dtype=jnp.int32)
                 % (NP - N))).reshape(NW, EPWP - EPW)
    src = jnp.concatenate([edge_index[0].reshape(NW, EPW), spad],
                          axis=1).reshape(NW, 2, NHALF, CH)
    dpad = (jnp.arange(NW * (EPWP - EPW), dtype=jnp.int32) % N).reshape(
        NW, EPWP - EPW)
    dst = jnp.concatenate([edge_index[1].reshape(NW, EPW), dpad],
                          axis=1).reshape(NW, 2, NHALF, CH)
    row = lambda v: v.reshape(1, -1)
    xp = jnp.pad(x, ((0, NP - N), (0, 0)))
    P1 = _segsum(xp, src, dst)
    res1 = _resk(xp, Wr1, row(br1))
    h1 = _comb1(P1, res1, W1, row(b1), row(g1), row(be1))
    P2 = _segsum(h1, src, dst)
    res2 = _resk(h1, Wr2, row(br2))
    return _comb2(P2, res2, W2, row(b2), row(g2), row(be2),
                  row(w_att), b_att.reshape(1, 1))


# speedup vs baseline: 3.3946x; 1.0754x over previous
import functools

import jax
import jax.numpy as jnp
from jax import lax
from jax.experimental import pallas as pl
from jax.experimental.pallas import tpu as pltpu
from jax.experimental.pallas import tpu_sc as plsc

N = 10000
E = 320000
D = 128

NC = 2
NS = 16
NW = NC * NS
EPW = E // NW
CH = 80
NHALF = 63
NCHUNK = 2 * NHALF
EPWP = NCHUNK * CH
NP = N + 128
WSTRIDE = 624
WROWS = 640

_SC_MESH = plsc.VectorSubcoreMesh(core_axis_name="c", subcore_axis_name="s")


@functools.partial(
    pl.kernel,
    out_type=jax.ShapeDtypeStruct((NC, N, D), jnp.float32),
    mesh=_SC_MESH,
    scratch_types=[
        pltpu.VMEM((NHALF, CH), jnp.int32),
        pltpu.VMEM((NHALF, CH), jnp.int32),
        pltpu.VMEM((CH, D), jnp.float32),
        pltpu.VMEM((CH, D), jnp.float32),
        pltpu.VMEM_SHARED((N, D), jnp.float32),
        pltpu.SemaphoreType.DMA,
        pltpu.SemaphoreType.DMA,
    ],
)
def _segsum(h_hbm, src_hbm, dst_hbm, out_hbm, src_all, dst_all, rows0, rows1,
            acc_sh, sem0, sem1):
    c = lax.axis_index("c")
    s = lax.axis_index("s")
    wid = s * NC + c
    row0 = jnp.minimum(s * WSTRIDE, N - WROWS)

    zero16 = jnp.zeros((16,), jnp.float32)

    def zrow(i, carry):
        for j in range(D // 16):
            rows0[i, pl.ds(j * 16, 16)] = zero16
        return carry

    lax.fori_loop(0, CH, zrow, 0)
    for k in range(WROWS // CH):
        pltpu.sync_copy(rows0, acc_sh.at[pl.ds(row0 + k * CH, CH)])
    plsc.subcore_barrier()

    def chunk_pair(i, carry):
        j0 = 2 * i
        d1 = pltpu.async_copy(h_hbm.at[src_all.at[j0 + 1]], rows1, sem1)
        pltpu.sync_copy(rows0, acc_sh.at[dst_all.at[j0]], add=True)
        d0 = pltpu.async_copy(h_hbm.at[src_all.at[j0 + 2]], rows0, sem0)
        d1.wait()
        pltpu.sync_copy(rows1, acc_sh.at[dst_all.at[j0 + 1]], add=True)
        d0.wait()
        return carry

    for half in range(2):
        pltpu.sync_copy(src_hbm.at[wid, half], src_all)
        pltpu.sync_copy(dst_hbm.at[wid, half], dst_all)
        pltpu.async_copy(h_hbm.at[src_all.at[0]], rows0, sem0).wait()
        lax.fori_loop(0, (NHALF - 1) // 2, chunk_pair, 0)
        pltpu.sync_copy(rows0, acc_sh.at[dst_all.at[NHALF - 1]], add=True)
    plsc.subcore_barrier()

    for k in range(WROWS // CH):
        pltpu.sync_copy(acc_sh.at[pl.ds(row0 + k * CH, CH)], rows0)
        pltpu.sync_copy(rows0, out_hbm.at[c, pl.ds(row0 + k * CH, CH)])


def _res_body(h_ref, Wr_ref, br_ref, o_ref):
    o_ref[...] = jnp.maximum(
        jnp.dot(h_ref[pl.ds(0, N), :], Wr_ref[...],
                preferred_element_type=jnp.float32) + br_ref[...], 0.0)


def _bn_combine(p_ref, res_ref, W_ref, b_ref, g_ref, be_ref):
    agg = p_ref[0] + p_ref[1]
    out = jnp.maximum(
        jnp.dot(agg, W_ref[...], preferred_element_type=jnp.float32)
        + b_ref[...], 0.0)
    out = out + res_ref[...]
    mu = jnp.mean(out, axis=0, keepdims=True)
    var = jnp.mean((out - mu) ** 2, axis=0, keepdims=True)
    return g_ref[...] * (out - mu) * lax.rsqrt(var + 1e-5) + be_ref[...]


def _comb1_body(p_ref, res_ref, W_ref, b_ref, g_ref, be_ref, o_ref):
    o_ref[pl.ds(0, N), :] = _bn_combine(p_ref, res_ref, W_ref, b_ref, g_ref,
                                        be_ref)
    o_ref[pl.ds(N, NP - N), :] = jnp.zeros((NP - N, D), jnp.float32)


def _comb2_body(p_ref, res_ref, W_ref, b_ref, g_ref, be_ref, watt_ref,
                batt_ref, o_ref):
    h2 = _bn_combine(p_ref, res_ref, W_ref, b_ref, g_ref, be_ref)
    logit = jnp.sum(h2 * watt_ref[...], axis=1, keepdims=True) + batt_ref[...]
    wgt = 1.0 / (1.0 + jnp.exp(-logit))
    hsum = jnp.sum(wgt * h2, axis=0, keepdims=True)
    hmax = jnp.max(h2, axis=0, keepdims=True)
    o_ref[...] = jnp.concatenate([hsum, hmax], axis=1)


_resk = pl.pallas_call(
    _res_body,
    out_shape=jax.ShapeDtypeStruct((N, D), jnp.float32),
)

_comb1 = pl.pallas_call(
    _comb1_body,
    out_shape=jax.ShapeDtypeStruct((NP, D), jnp.float32),
)

_comb2 = pl.pallas_call(
    _comb2_body,
    out_shape=jax.ShapeDtypeStruct((1, 2 * D), jnp.float32),
)


def kernel(x, edge_index, W1, b1, Wr1, br1, g1, be1, W2, b2, Wr2, br2, g2,
           be2, w_att, b_att):
    spad = (N + (jnp.arange(NW * (EPWP - EPW), dtype=jnp.int32)
                 % (NP - N))).reshape(NW, EPWP - EPW)
    src = jnp.concatenate([edge_index[0].reshape(NW, EPW), spad],
                          axis=1).reshape(NW, 2, NHALF, CH)
    dpad = (jnp.arange(NW * (EPWP - EPW), dtype=jnp.int32) % N).reshape(
        NW, EPWP - EPW)
    dst = jnp.concatenate([edge_index[1].reshape(NW, EPW), dpad],
                          axis=1).reshape(NW, 2, NHALF, CH)
    row = lambda v: v.reshape(1, -1)
    xp = jnp.pad(x, ((0, NP - N), (0, 0)))
    P1 = _segsum(xp, src, dst)
    res1 = _resk(xp, Wr1, row(br1))
    h1 = _comb1(P1, res1, W1, row(b1), row(g1), row(be1))
    P2 = _segsum(h1, src, dst)
    res2 = _resk(h1, Wr2, row(br2))
    return _comb2(P2, res2, W2, row(b2), row(g2), row(be2),
                  row(w_att), b_att.reshape(1, 1))
